# Initial kernel scaffold; baseline (speedup 1.0000x reference)
import jax
import jax.numpy as jnp
from jax.experimental import pallas as pl


def kernel(b, Wg, bg, Wth, bth, Wph, bph, Ww, bw):
    raise NotImplementedError("write your pallas kernel here")


# speedup vs baseline: 4.6423x; 4.6423x over previous
import numpy as np
import jax
import jax.numpy as jnp
from jax.experimental import pallas as pl
from jax.experimental.pallas import tpu as pltpu

KSIZE = 7
STRIDE_Q = 4
SCALE = 10.0
TOPK = 100
T, C_IN, H, W = 4, 64, 64, 64
C_INT = 16
NQ_SIDE = 16
NK_SIDE = 64
NQ = NQ_SIDE * NQ_SIDE
NK = NK_SIDE * NK_SIDE
D = C_INT * KSIZE * KSIZE
PT_Q = 1
N_ITERS = 40


def _fold_constants():
    mcols = np.zeros((KSIZE, NQ_SIDE, W), np.float32)
    for dw in range(KSIZE):
        for qj in range(NQ_SIDE):
            s = STRIDE_Q * qj + dw - PT_Q
            if 0 <= s < W:
                mcols[dw, qj, s] = 1.0
    rt = np.zeros((H, KSIZE * NQ_SIDE), np.float32)
    for dh in range(KSIZE):
        for qi in range(NQ_SIDE):
            r = STRIDE_Q * qi + dh - PT_Q
            if 0 <= r < H:
                rt[r, dh * NQ_SIDE + qi] = 1.0
    cov = np.zeros((H,), np.float32)
    for r in range(H):
        rp = r + PT_Q
        for qi in range(NQ_SIDE):
            if 0 <= rp - STRIDE_Q * qi < KSIZE:
                cov[r] += 1.0
    inv_cnt = (1.0 / np.outer(cov, cov)).astype(np.float32)
    return mcols, rt, inv_cnt


_MCOLS, _RT, _INV_CNT = _fold_constants()


def _proj_body(b_ref, w_ref, bias_ref, out_ref):
    out_ref[0] = (
        jnp.dot(w_ref[...], b_ref[0], preferred_element_type=jnp.float32)
        + bias_ref[...]
    )


def _wts_body(qt_ref, kpat_ref, wts_ref):
    qt = qt_ref[0]
    kpat = kpat_ref[0]
    d = jnp.dot(kpat, qt, preferred_element_type=jnp.float32)

    m = jnp.max(d, axis=0, keepdims=True)
    lo0 = jnp.min(d, axis=0, keepdims=True)

    def body(_, carry):
        lo, hi = carry
        mid = 0.5 * (lo + hi)
        cnt = jnp.sum((d >= mid).astype(jnp.float32), axis=0, keepdims=True)
        take = cnt >= TOPK
        return jnp.where(take, mid, lo), jnp.where(take, hi, mid)

    lo, _ = jax.lax.fori_loop(0, N_ITERS, body, (lo0, m))

    e = jnp.where(d >= lo, jnp.exp((d - m) * SCALE), 0.0)
    wts_ref[0] = e / jnp.sum(e, axis=0, keepdims=True)


def _fold_body(wts_ref, vpt_ref, b_ref, mcols_ref, rt_ref,
               icnt_ref, ww_ref, bw_ref, out_ref):
    zt = jnp.dot(vpt_ref[0], wts_ref[0],
                 preferred_element_type=jnp.float32)

    z5 = zt.reshape(C_INT, KSIZE, KSIZE, NQ_SIDE, NQ_SIDE)
    a = jnp.zeros((C_INT * KSIZE * NQ_SIDE, W), jnp.float32)
    for dw in range(KSIZE):
        s = z5[:, :, dw, :, :].reshape(C_INT * KSIZE * NQ_SIDE, NQ_SIDE)
        a = a + jnp.dot(s, mcols_ref[dw], preferred_element_type=jnp.float32)
    a3 = a.reshape(C_INT, KSIZE * NQ_SIDE, W)
    rt = rt_ref[...]
    icnt = icnt_ref[...]
    ys = [jnp.dot(rt, a3[c], preferred_element_type=jnp.float32) * icnt
          for c in range(C_INT)]
    y = jnp.stack(ys, 0).reshape(C_INT, H * W)

    out_ref[0] = (
        jnp.dot(ww_ref[...], y, preferred_element_type=jnp.float32)
        + bw_ref[...]
        + b_ref[0]
    )


def _unfold(xp, stride):
    n_h = (xp.shape[2] - KSIZE) // stride + 1
    n_w = (xp.shape[3] - KSIZE) // stride + 1
    idx_r = (jnp.arange(n_h) * stride)[:, None] + jnp.arange(KSIZE)[None, :]
    idx_c = (jnp.arange(n_w) * stride)[:, None] + jnp.arange(KSIZE)[None, :]
    p = xp[:, :, idx_r][:, :, :, :, idx_c]
    p = jnp.transpose(p, (0, 2, 4, 1, 3, 5)).reshape(
        xp.shape[0], n_h * n_w, xp.shape[1] * KSIZE * KSIZE)
    return p


def kernel(b, Wg, bg, Wth, bth, Wph, bph, Ww, bw):
    bf = b.reshape(T, C_IN, H * W)
    wcat = jnp.concatenate([Wg, Wth, Wph], axis=0)
    bcat = jnp.concatenate([bg, bth, bph]).reshape(-1, 1)

    proj = pl.pallas_call(
        _proj_body,
        grid=(T,),
        in_specs=[
            pl.BlockSpec((1, C_IN, H * W), lambda t: (t, 0, 0)),
            pl.BlockSpec((3 * C_INT, C_IN), lambda t: (0, 0)),
            pl.BlockSpec((3 * C_INT, 1), lambda t: (0, 0)),
        ],
        out_specs=pl.BlockSpec((1, 3 * C_INT, H * W), lambda t: (t, 0, 0)),
        out_shape=jax.ShapeDtypeStruct((T, 3 * C_INT, H * W), jnp.float32),
    )(bf, wcat, bcat)

    b1 = proj[:, 0:C_INT].reshape(T, C_INT, H, W)
    b2 = proj[:, C_INT:2 * C_INT].reshape(T, C_INT, H, W)
    b3 = proj[:, 2 * C_INT:].reshape(T, C_INT, H, W)

    qp = jnp.pad(b1, ((0, 0), (0, 0), (1, 2), (1, 2)))
    kp = jnp.pad(b3, ((0, 0), (0, 0), (3, 3), (3, 3)))
    vp = jnp.pad(b2, ((0, 0), (0, 0), (3, 3), (3, 3)))

    qt = _unfold(qp, STRIDE_Q).transpose(0, 2, 1)
    kpat = _unfold(kp, 1)
    vpt = _unfold(vp, 1).transpose(0, 2, 1)

    wts = pl.pallas_call(
        _wts_body,
        grid=(T,),
        in_specs=[
            pl.BlockSpec((1, D, NQ), lambda t: (t, 0, 0)),
            pl.BlockSpec((1, NK, D), lambda t: (t, 0, 0)),
        ],
        out_specs=pl.BlockSpec((1, NK, NQ), lambda t: (t, 0, 0)),
        out_shape=jax.ShapeDtypeStruct((T, NK, NQ), jnp.float32),
    )(qt, kpat)

    out = pl.pallas_call(
        _fold_body,
        grid=(T,),
        in_specs=[
            pl.BlockSpec((1, NK, NQ), lambda t: (t, 0, 0)),
            pl.BlockSpec((1, D, NK), lambda t: (t, 0, 0)),
            pl.BlockSpec((1, C_IN, H * W), lambda t: (t, 0, 0)),
            pl.BlockSpec((KSIZE, NQ_SIDE, W), lambda t: (0, 0, 0)),
            pl.BlockSpec((H, KSIZE * NQ_SIDE), lambda t: (0, 0)),
            pl.BlockSpec((H, W), lambda t: (0, 0)),
            pl.BlockSpec((C_IN, C_INT), lambda t: (0, 0)),
            pl.BlockSpec((C_IN, 1), lambda t: (0, 0)),
        ],
        out_specs=pl.BlockSpec((1, C_IN, H * W), lambda t: (t, 0, 0)),
        out_shape=jax.ShapeDtypeStruct((T, C_IN, H * W), jnp.float32),
    )(wts, vpt, bf, _MCOLS, _RT, _INV_CNT, Ww, bw.reshape(-1, 1))

    return out.reshape(T, C_IN, H, W)
